# baseline (speedup 1.0000x reference)
import functools

import jax
import jax.numpy as jnp
from jax import lax
from jax.experimental import pallas as pl
from jax.experimental.pallas import tpu as pltpu
from jax.experimental.pallas import tpu_sc as plsc

N_USERS = 40000
N_ITEMS = 10000
N = N_USERS + N_ITEMS
E = 320000
D = 128

NCHUNK = 8
CHUNK = 6272
NPAD = NCHUNK * CHUNK
DUMMY = CHUNK
ACC_ROWS = 6400
BLK = 128
NTILES = 16
PROD = 32
EPT = E // PROD
IB = 2000
CAPB = (EPT + BLK - 1) // BLK
BLKW = 3 * BLK
SLOT_W = CAPB * BLKW
ZROWS = ACC_ROWS // BLK
NWB = CHUNK // BLK

_mesh = plsc.VectorSubcoreMesh(core_axis_name="c", subcore_axis_name="s")


@functools.partial(
    pl.kernel,
    out_type=(
        jax.ShapeDtypeStruct((NCHUNK * PROD * SLOT_W,), jnp.int32),
        jax.ShapeDtypeStruct((PROD * 16,), jnp.int32),
    ),
    mesh=_mesh,
    compiler_params=pltpu.CompilerParams(needs_layout_passes=False),
    scratch_types=[
        pltpu.VMEM((IB,), jnp.int32),
        pltpu.VMEM((IB,), jnp.int32),
        pltpu.VMEM((IB,), jnp.float32),
        [pltpu.VMEM((144,), jnp.int32) for _ in range(NCHUNK)],
        [pltpu.VMEM((144,), jnp.int32) for _ in range(NCHUNK)],
        [pltpu.VMEM((144,), jnp.int32) for _ in range(NCHUNK)],
        pltpu.VMEM((16,), jnp.int32),
    ],
)
def _bin(src_hbm, dst_hbm, w_hbm, edata_hbm, cnt_hbm,
         srcv, dstv, wvv, stD, stS, stW, cstg):
    cid = lax.axis_index("c")
    sid = lax.axis_index("s")
    t = cid * NTILES + sid
    iota = lax.iota(jnp.int32, 16)

    def vec_body(vi, carry):
        off = vi * 16
        s = srcv[pl.ds(off, 16)]
        d = dstv[pl.ds(off, 16)]
        wb = lax.bitcast_convert_type(wvv[pl.ds(off, 16)], jnp.int32)
        bkt = (s >= CHUNK).astype(jnp.int32)
        for k in range(2, NCHUNK):
            bkt = bkt + (s >= k * CHUNK).astype(jnp.int32)
        loc = s - bkt * CHUNK
        new_carry = []
        for bk in range(NCHUNK):
            cur = carry[bk]
            blk = carry[NCHUNK + bk]
            m = bkt == bk
            plsc.store_compressed(stD[bk].at[pl.ds(cur, 16)], d, mask=m)
            plsc.store_compressed(stS[bk].at[pl.ds(cur, 16)], loc, mask=m)
            plsc.store_compressed(stW[bk].at[pl.ds(cur, 16)], wb, mask=m)
            cur = cur + plsc.all_reduce_population_count(m)[0]
            full = cur >= BLK

            @pl.when(full)
            def _flush(bk=bk, blk=blk):
                addr = (bk * PROD + t) * SLOT_W + blk * BLKW
                pltpu.sync_copy(stD[bk].at[pl.ds(0, BLK)],
                                edata_hbm.at[pl.ds(addr, BLK)])
                pltpu.sync_copy(stS[bk].at[pl.ds(0, BLK)],
                                edata_hbm.at[pl.ds(addr + BLK, BLK)])
                pltpu.sync_copy(stW[bk].at[pl.ds(0, BLK)],
                                edata_hbm.at[pl.ds(addr + 2 * BLK, BLK)])
                stD[bk][pl.ds(0, 16)] = stD[bk][pl.ds(BLK, 16)]
                stS[bk][pl.ds(0, 16)] = stS[bk][pl.ds(BLK, 16)]
                stW[bk][pl.ds(0, 16)] = stW[bk][pl.ds(BLK, 16)]

            new_carry.append(jnp.where(full, cur - BLK, cur))
            carry = carry[:NCHUNK + bk] + (blk + full.astype(jnp.int32),) \
                + carry[NCHUNK + bk + 1:]
        return tuple(new_carry) + carry[NCHUNK:]

    def ib_body(ib, carry):
        base_e = t * EPT + ib * IB
        pltpu.sync_copy(src_hbm.at[pl.ds(base_e, IB)], srcv)
        pltpu.sync_copy(dst_hbm.at[pl.ds(base_e, IB)], dstv)
        pltpu.sync_copy(w_hbm.at[pl.ds(base_e, IB)], wvv)
        return lax.fori_loop(0, IB // 16, vec_body, carry)

    carry = lax.fori_loop(0, EPT // IB, ib_body,
                          (jnp.int32(0),) * NCHUNK + (jnp.int32(0),) * NCHUNK)

    final = []
    for bk in range(NCHUNK):
        cur = carry[bk]
        blk = carry[NCHUNK + bk]
        for j in range(8):
            posm = (iota + j * 16) < cur
            sl = pl.ds(j * 16, 16)
            stD[bk][sl] = jnp.where(posm, stD[bk][sl], 0)
            stS[bk][sl] = jnp.where(posm, stS[bk][sl], DUMMY)
            stW[bk][sl] = jnp.where(posm, stW[bk][sl], 0)

        @pl.when(cur > 0)
        def _tail(bk=bk, blk=blk):
            addr = (bk * PROD + t) * SLOT_W + blk * BLKW
            pltpu.sync_copy(stD[bk].at[pl.ds(0, BLK)],
                            edata_hbm.at[pl.ds(addr, BLK)])
            pltpu.sync_copy(stS[bk].at[pl.ds(0, BLK)],
                            edata_hbm.at[pl.ds(addr + BLK, BLK)])
            pltpu.sync_copy(stW[bk].at[pl.ds(0, BLK)],
                            edata_hbm.at[pl.ds(addr + 2 * BLK, BLK)])

        final.append(blk + (cur > 0).astype(jnp.int32))

    cvec = jnp.zeros((16,), jnp.int32)
    for bk in range(NCHUNK):
        cvec = jnp.where(iota == bk, final[bk], cvec)
    cstg[pl.ds(0, 16)] = cvec
    pltpu.sync_copy(cstg, cnt_hbm.at[pl.ds(t * 16, 16)])


@functools.partial(
    pl.kernel,
    out_type=jax.ShapeDtypeStruct((NPAD, D), jnp.float32),
    mesh=_mesh,
    compiler_params=pltpu.CompilerParams(needs_layout_passes=False),
    scratch_types=[
        pltpu.VMEM_SHARED((ACC_ROWS, D), jnp.float32),
        pltpu.VMEM((BLKW,), jnp.int32),
        pltpu.VMEM((BLKW,), jnp.int32),
        pltpu.VMEM((1, BLK), jnp.int32),
        pltpu.VMEM((1, BLK), jnp.int32),
        pltpu.VMEM((BLK, D), jnp.float32),
        pltpu.VMEM((BLK, D), jnp.float32),
        pltpu.VMEM((32,), jnp.int32),
        pltpu.SemaphoreType.DMA,
        pltpu.SemaphoreType.DMA,
    ],
)
def _spmm(x_hbm, edata_hbm, cnt_hbm, out_hbm,
          acc, edv0, edv1, sidx0, sidx1, rows0, rows1, cntv, sem0, sem1):
    cid = lax.axis_index("c")
    sid = lax.axis_index("s")

    pltpu.sync_copy(cnt_hbm.at[pl.ds(sid * 32, 32)], cntv)
    ca = cntv[pl.ds(0, 16)]
    cb = cntv[pl.ds(16, 16)]

    zero16 = jnp.zeros((16,), jnp.float32)

    def zrow(k, carry):
        for c8 in range(8):
            rows0[k, pl.ds(c8 * 16, 16)] = zero16
        return carry

    def make_mul(edv, rows):
        def mul_j8(j8, carry):
            w16 = lax.bitcast_convert_type(edv[pl.ds(2 * BLK + j8 * 16, 16)],
                                           jnp.float32)
            for e in range(16):
                k = j8 * 16 + e
                wk = w16[e]
                for c8 in range(8):
                    sl = pl.ds(c8 * 16, 16)
                    rows[k, sl] = rows[k, sl] * wk
            return carry
        return mul_j8

    mul0 = make_mul(edv0, rows0)
    mul1 = make_mul(edv1, rows1)

    for b in range(NCHUNK):

        @pl.when(cid == b // (NCHUNK // 2))
        def _process(b=b):
            base = b * CHUNK
            lax.fori_loop(0, BLK, zrow, 0)

            def z_body(i, carry):
                zb = sid + i * NTILES
                pltpu.sync_copy(rows0, acc.at[pl.ds(zb * BLK, BLK)])
                return carry

            lax.fori_loop(0, (ZROWS - 1 - sid) // NTILES + 1, z_body, 0)
            plsc.subcore_barrier()

            n0 = ca[b]
            n1 = cb[b]
            nt = n0 + n1
            sbase0 = (b * PROD + 2 * sid) * SLOT_W
            sbase1 = (b * PROD + 2 * sid + 1) * SLOT_W

            def fetch(i, edv, sidx, rows, sem):
                a = jnp.where(i < n0, sbase0 + i * BLKW,
                              sbase1 + (i - n0) * BLKW)
                pltpu.sync_copy(edata_hbm.at[pl.ds(a, BLKW)], edv)
                for j in range(8):
                    sidx[0, pl.ds(j * 16, 16)] = edv[pl.ds(BLK + j * 16, 16)]
                pltpu.async_copy(x_hbm.at[edv.at[pl.ds(0, BLK)]], rows, sem)

            def drain(rows, sem):
                pltpu.make_async_copy(x_hbm.at[pl.ds(0, BLK)], rows,
                                      sem).wait()

            @pl.when(nt > 0)
            def _prime():
                fetch(jnp.int32(0), edv0, sidx0, rows0, sem0)

            def grp(g, carry):
                i1 = 2 * g + 1

                @pl.when(i1 < nt)
                def _f1():
                    fetch(i1, edv1, sidx1, rows1, sem1)

                drain(rows0, sem0)
                lax.fori_loop(0, 8, mul0, 0)
                pltpu.sync_copy(rows0, acc.at[sidx0.at[0]], add=True)

                @pl.when(i1 + 1 < nt)
                def _f0():
                    fetch(i1 + 1, edv0, sidx0, rows0, sem0)

                @pl.when(i1 < nt)
                def _p1():
                    drain(rows1, sem1)
                    lax.fori_loop(0, 8, mul1, 0)
                    pltpu.sync_copy(rows1, acc.at[sidx1.at[0]], add=True)

                return carry

            lax.fori_loop(0, (nt + 1) // 2, grp, 0)
            plsc.subcore_barrier()

            def wr_body(i, carry):
                r0 = (sid + i * NTILES) * BLK
                pltpu.sync_copy(acc.at[pl.ds(r0, BLK)], rows0)
                pltpu.sync_copy(rows0, out_hbm.at[pl.ds(base + r0, BLK)])
                return carry

            lax.fori_loop(0, (NWB - 1 - sid) // NTILES + 1, wr_body, 0)
            plsc.subcore_barrier()


def _mean4_body(a_ref, b_ref, c_ref, d_ref, o_ref):
    o_ref[...] = (a_ref[...] + b_ref[...] + c_ref[...] + d_ref[...]) * 0.25


_MEAN_BLKS = 16
_MEAN_ROWS = NPAD // _MEAN_BLKS


def _mean4(a, b, c, d):
    spec = pl.BlockSpec((_MEAN_ROWS, D), lambda i: (i, 0))
    return pl.pallas_call(
        _mean4_body,
        out_shape=jax.ShapeDtypeStruct((NPAD, D), jnp.float32),
        grid=(_MEAN_BLKS,),
        in_specs=[spec, spec, spec, spec],
        out_specs=spec,
    )(a, b, c, d)


def kernel(all_emb, edge_index, edge_weight):
    src = edge_index[0]
    dst = edge_index[1]
    x0 = jnp.pad(all_emb, ((0, NPAD - N), (0, 0)))
    edata, cnts = _bin(src, dst, edge_weight)
    x1 = _spmm(x0, edata, cnts)
    x2 = _spmm(x1, edata, cnts)
    x3 = _spmm(x2, edata, cnts)
    out = _mean4(x0, x1, x2, x3)
    return out[:N_USERS], out[N_USERS:N]


# speedup vs baseline: 1.2714x; 1.2714x over previous
import functools

import jax
import jax.numpy as jnp
from jax import lax
from jax.experimental import pallas as pl
from jax.experimental.pallas import tpu as pltpu
from jax.experimental.pallas import tpu_sc as plsc

N_USERS = 40000
N_ITEMS = 10000
N = N_USERS + N_ITEMS
E = 320000
D = 128
NCHUNK = 4
CHUNK = 12544
NPAD = NCHUNK * CHUNK
DUMMY = CHUNK
ACC_ROWS = 12672
BLK = 128
NTILES = 16
PROD = 32
EPT = E // PROD
IB = 2000
CAPB = (EPT + BLK - 1) // BLK
BLKW = 3 * BLK
SLOT_W = CAPB * BLKW
ZROWS = ACC_ROWS // BLK
NWB = CHUNK // BLK

_mesh = plsc.VectorSubcoreMesh(core_axis_name="c", subcore_axis_name="s")


@functools.partial(
    pl.kernel,
    out_type=(
        jax.ShapeDtypeStruct((NCHUNK * PROD * SLOT_W,), jnp.int32),
        jax.ShapeDtypeStruct((PROD * 16,), jnp.int32),
    ),
    mesh=_mesh,
    compiler_params=pltpu.CompilerParams(needs_layout_passes=False),
    scratch_types=[
        pltpu.VMEM((IB,), jnp.int32),
        pltpu.VMEM((IB,), jnp.int32),
        pltpu.VMEM((IB,), jnp.float32),
        [pltpu.VMEM((144,), jnp.int32) for _ in range(NCHUNK)],
        [pltpu.VMEM((144,), jnp.int32) for _ in range(NCHUNK)],
        [pltpu.VMEM((144,), jnp.int32) for _ in range(NCHUNK)],
        pltpu.VMEM((16,), jnp.int32),
    ],
)
def _bin(src_hbm, dst_hbm, w_hbm, edata_hbm, cnt_hbm,
         srcv, dstv, wvv, stD, stS, stW, cstg):
    cid = lax.axis_index("c")
    sid = lax.axis_index("s")
    t = cid * NTILES + sid
    iota = lax.iota(jnp.int32, 16)

    def vec_body(vi, carry):
        off = vi * 16
        s = srcv[pl.ds(off, 16)]
        d = dstv[pl.ds(off, 16)]
        wb = lax.bitcast_convert_type(wvv[pl.ds(off, 16)], jnp.int32)
        bkt = ((s >= CHUNK).astype(jnp.int32)
               + (s >= 2 * CHUNK).astype(jnp.int32)
               + (s >= 3 * CHUNK).astype(jnp.int32))
        loc = s - bkt * CHUNK
        new_carry = []
        for bk in range(NCHUNK):
            cur = carry[bk]
            blk = carry[NCHUNK + bk]
            m = bkt == bk
            plsc.store_compressed(stD[bk].at[pl.ds(cur, 16)], d, mask=m)
            plsc.store_compressed(stS[bk].at[pl.ds(cur, 16)], loc, mask=m)
            plsc.store_compressed(stW[bk].at[pl.ds(cur, 16)], wb, mask=m)
            cur = cur + plsc.all_reduce_population_count(m)[0]
            full = cur >= BLK

            @pl.when(full)
            def _flush(bk=bk, blk=blk):
                addr = (bk * PROD + t) * SLOT_W + blk * BLKW
                pltpu.sync_copy(stD[bk].at[pl.ds(0, BLK)],
                                edata_hbm.at[pl.ds(addr, BLK)])
                pltpu.sync_copy(stS[bk].at[pl.ds(0, BLK)],
                                edata_hbm.at[pl.ds(addr + BLK, BLK)])
                pltpu.sync_copy(stW[bk].at[pl.ds(0, BLK)],
                                edata_hbm.at[pl.ds(addr + 2 * BLK, BLK)])
                stD[bk][pl.ds(0, 16)] = stD[bk][pl.ds(BLK, 16)]
                stS[bk][pl.ds(0, 16)] = stS[bk][pl.ds(BLK, 16)]
                stW[bk][pl.ds(0, 16)] = stW[bk][pl.ds(BLK, 16)]

            new_carry.append(jnp.where(full, cur - BLK, cur))
            carry = carry[:NCHUNK + bk] + (blk + full.astype(jnp.int32),) \
                + carry[NCHUNK + bk + 1:]
        return tuple(new_carry) + carry[NCHUNK:]

    def ib_body(ib, carry):
        base_e = t * EPT + ib * IB
        pltpu.sync_copy(src_hbm.at[pl.ds(base_e, IB)], srcv)
        pltpu.sync_copy(dst_hbm.at[pl.ds(base_e, IB)], dstv)
        pltpu.sync_copy(w_hbm.at[pl.ds(base_e, IB)], wvv)
        return lax.fori_loop(0, IB // 16, vec_body, carry)

    carry = lax.fori_loop(0, EPT // IB, ib_body,
                          (jnp.int32(0),) * NCHUNK + (jnp.int32(0),) * NCHUNK)

    final = []
    for bk in range(NCHUNK):
        cur = carry[bk]
        blk = carry[NCHUNK + bk]
        for j in range(8):
            posm = (iota + j * 16) < cur
            sl = pl.ds(j * 16, 16)
            stD[bk][sl] = jnp.where(posm, stD[bk][sl], 0)
            stS[bk][sl] = jnp.where(posm, stS[bk][sl], DUMMY)
            stW[bk][sl] = jnp.where(posm, stW[bk][sl], 0)

        @pl.when(cur > 0)
        def _tail(bk=bk, blk=blk):
            addr = (bk * PROD + t) * SLOT_W + blk * BLKW
            pltpu.sync_copy(stD[bk].at[pl.ds(0, BLK)],
                            edata_hbm.at[pl.ds(addr, BLK)])
            pltpu.sync_copy(stS[bk].at[pl.ds(0, BLK)],
                            edata_hbm.at[pl.ds(addr + BLK, BLK)])
            pltpu.sync_copy(stW[bk].at[pl.ds(0, BLK)],
                            edata_hbm.at[pl.ds(addr + 2 * BLK, BLK)])

        final.append(blk + (cur > 0).astype(jnp.int32))

    cvec = jnp.zeros((16,), jnp.int32)
    for bk in range(NCHUNK):
        cvec = jnp.where(iota == bk, final[bk], cvec)
    cstg[pl.ds(0, 16)] = cvec
    pltpu.sync_copy(cstg, cnt_hbm.at[pl.ds(t * 16, 16)])


@functools.partial(
    pl.kernel,
    out_type=jax.ShapeDtypeStruct((NPAD, D), jnp.float32),
    mesh=_mesh,
    compiler_params=pltpu.CompilerParams(needs_layout_passes=False),
    scratch_types=[
        pltpu.VMEM_SHARED((ACC_ROWS, D), jnp.float32),
        pltpu.VMEM((BLKW,), jnp.int32),
        pltpu.VMEM((1, BLK), jnp.int32),
        pltpu.VMEM((BLK, D), jnp.float32),
        pltpu.VMEM((32,), jnp.int32),
        pltpu.SemaphoreType.DMA,
    ],
)
def _spmm(x_hbm, edata_hbm, cnt_hbm, out_hbm,
          acc, edv, sidx, rows, cntv, sem):
    cid = lax.axis_index("c")
    sid = lax.axis_index("s")

    pltpu.sync_copy(cnt_hbm.at[pl.ds(sid * 32, 32)], cntv)
    ca = cntv[pl.ds(0, 16)]
    cb = cntv[pl.ds(16, 16)]

    zero16 = jnp.zeros((16,), jnp.float32)

    def zrow(k, carry):
        for c8 in range(8):
            rows[k, pl.ds(c8 * 16, 16)] = zero16
        return carry

    def mul_j8(j8, carry):
        w16 = lax.bitcast_convert_type(edv[pl.ds(2 * BLK + j8 * 16, 16)],
                                       jnp.float32)
        for e in range(16):
            k = j8 * 16 + e
            wk = w16[e]
            for c8 in range(8):
                sl = pl.ds(c8 * 16, 16)
                rows[k, sl] = rows[k, sl] * wk
        return carry

    def make_blk_body(sbase):
        def blk_body(i, carry):
            addr = sbase + i * BLKW
            pltpu.sync_copy(edata_hbm.at[pl.ds(addr, BLKW)], edv)
            for j in range(8):
                sidx[0, pl.ds(j * 16, 16)] = edv[pl.ds(BLK + j * 16, 16)]
            pltpu.async_copy(x_hbm.at[edv.at[pl.ds(0, BLK)]], rows, sem).wait()
            lax.fori_loop(0, 8, mul_j8, 0)
            pltpu.sync_copy(rows, acc.at[sidx.at[0]], add=True)
            return carry
        return blk_body

    for b in range(NCHUNK):

        @pl.when(cid == b // 2)
        def _process(b=b):
            base = b * CHUNK
            lax.fori_loop(0, BLK, zrow, 0)

            def z_body(i, carry):
                zb = sid + i * NTILES
                pltpu.sync_copy(rows, acc.at[pl.ds(zb * BLK, BLK)])
                return carry

            lax.fori_loop(0, (ZROWS - 1 - sid) // NTILES + 1, z_body, 0)
            plsc.subcore_barrier()

            for p in range(2):
                tprod = 2 * sid + p
                sbase = (b * PROD + tprod) * SLOT_W
                nb = (ca if p == 0 else cb)[b]
                lax.fori_loop(0, nb, make_blk_body(sbase), 0)
            plsc.subcore_barrier()

            def wr_body(i, carry):
                r0 = (sid + i * NTILES) * BLK
                pltpu.sync_copy(acc.at[pl.ds(r0, BLK)], rows)
                pltpu.sync_copy(rows, out_hbm.at[pl.ds(base + r0, BLK)])
                return carry

            lax.fori_loop(0, (NWB - 1 - sid) // NTILES + 1, wr_body, 0)
            plsc.subcore_barrier()


def _mean4_body(a_ref, b_ref, c_ref, d_ref, o_ref):
    o_ref[...] = (a_ref[...] + b_ref[...] + c_ref[...] + d_ref[...]) * 0.25


_MEAN_BLKS = 16
_MEAN_ROWS = NPAD // _MEAN_BLKS


def _mean4(a, b, c, d):
    spec = pl.BlockSpec((_MEAN_ROWS, D), lambda i: (i, 0))
    return pl.pallas_call(
        _mean4_body,
        out_shape=jax.ShapeDtypeStruct((NPAD, D), jnp.float32),
        grid=(_MEAN_BLKS,),
        in_specs=[spec, spec, spec, spec],
        out_specs=spec,
    )(a, b, c, d)


def kernel(all_emb, edge_index, edge_weight):
    src = edge_index[0]
    dst = edge_index[1]
    x0 = jnp.pad(all_emb, ((0, NPAD - N), (0, 0)))
    edata, cnts = _bin(src, dst, edge_weight)
    x1 = _spmm(x0, edata, cnts)
    x2 = _spmm(x1, edata, cnts)
    x3 = _spmm(x2, edata, cnts)
    out = _mean4(x0, x1, x2, x3)
    return out[:N_USERS], out[N_USERS:N]
